# Initial kernel scaffold; baseline (speedup 1.0000x reference)
import functools

import jax
import jax.numpy as jnp
from jax import lax
from jax.experimental import pallas as pl
from jax.experimental.pallas import tpu as pltpu
from jax.experimental.pallas import tpu_sc as plsc

N_NODES = 10000
N_EDGES = 160000
D_IN = 256
D_OUT = 256
H = 128
NC = 2
NS = 16
LANES = 128
PER_TILE = -(-N_EDGES // (NS * LANES)) * LANES
CHUNKS = PER_TILE // LANES
E_PAD = PER_TILE * NS
NP = N_NODES + 8
ROWS_PER_TILE = N_NODES // NS


def _sc_scatter_body(tbl, gidx, didx, zrs, out0, out1, acc, gi_v, di_v, rows, sem):
    c = lax.axis_index("c")
    s = lax.axis_index("s")
    w = c * NS + s
    pltpu.sync_copy(gidx.at[w], gi_v)
    pltpu.sync_copy(didx.at[s], di_v)
    pltpu.sync_copy(zrs, acc.at[pl.ds(s * ROWS_PER_TILE, ROWS_PER_TILE)])
    plsc.subcore_barrier()

    def chunk(j, carry):
        pltpu.async_copy(tbl.at[gi_v.at[j]], rows, sem).wait()
        pltpu.sync_copy(rows, acc.at[di_v.at[j]], add=True)
        return carry

    lax.fori_loop(0, CHUNKS, chunk, 0)
    plsc.subcore_barrier()

    sl = pl.ds(s * ROWS_PER_TILE, ROWS_PER_TILE)

    @pl.when(c == 0)
    def _():
        pltpu.sync_copy(acc.at[sl], out0.at[sl])

    @pl.when(c == 1)
    def _():
        pltpu.sync_copy(acc.at[sl], out1.at[sl])


@functools.partial(
    pl.kernel,
    out_type=(
        jax.ShapeDtypeStruct((N_NODES, H), jnp.float32),
        jax.ShapeDtypeStruct((N_NODES, H), jnp.float32),
    ),
    mesh=plsc.VectorSubcoreMesh(core_axis_name="c", subcore_axis_name="s"),
    scratch_types=[
        pltpu.VMEM_SHARED((N_NODES, H), jnp.float32),
        pltpu.VMEM((CHUNKS, LANES), jnp.int32),
        pltpu.VMEM((CHUNKS, LANES), jnp.int32),
        pltpu.VMEM((LANES, H), jnp.float32),
        pltpu.SemaphoreType.DMA,
    ],
)
def _sc_scatter(tbl, gidx, didx, zrs, out0, out1, acc, gi_v, di_v, rows, sem):
    _sc_scatter_body(tbl, gidx, didx, zrs, out0, out1, acc, gi_v, di_v, rows, sem)


def _mm_body(h0_ref, h1_ref, x_ref, wl0_ref, wl1_ref, wp_ref, b_ref, o_ref):
    dn = (((1,), (1,)), ((), ()))
    acc = lax.dot_general(h0_ref[...], wl0_ref[...], dn,
                          preferred_element_type=jnp.float32)
    acc += lax.dot_general(h1_ref[...], wl1_ref[...], dn,
                           preferred_element_type=jnp.float32)
    acc += lax.dot_general(x_ref[...], wp_ref[...], dn,
                           preferred_element_type=jnp.float32)
    o_ref[...] = acc + b_ref[...]


def _tc_linear(h0, h1, x, wl0, wl1, wp, b):
    blk = 1000
    grid = (N_NODES // blk,)
    return pl.pallas_call(
        _mm_body,
        grid=grid,
        in_specs=[
            pl.BlockSpec((blk, H), lambda i: (i, 0)),
            pl.BlockSpec((blk, H), lambda i: (i, 0)),
            pl.BlockSpec((blk, D_IN), lambda i: (i, 0)),
            pl.BlockSpec((D_OUT, H), lambda i: (0, 0)),
            pl.BlockSpec((D_OUT, H), lambda i: (0, 0)),
            pl.BlockSpec((D_OUT, D_IN), lambda i: (0, 0)),
            pl.BlockSpec((1, D_OUT), lambda i: (0, 0)),
        ],
        out_specs=pl.BlockSpec((blk, D_OUT), lambda i: (i, 0)),
        out_shape=jax.ShapeDtypeStruct((N_NODES, D_OUT), jnp.float32),
    )(h0, h1, x, wl0, wl1, wp, b)


def kernel(input_feat, edge_index, W_lin, b_lin, W_loop, b_loop, bias):
    src = edge_index[0].astype(jnp.int32)
    dst = edge_index[1].astype(jnp.int32)
    pad = E_PAD - N_EDGES
    src_p = jnp.concatenate([src, jnp.full((pad,), N_NODES, jnp.int32)])
    dst_p = jnp.concatenate([dst, jnp.zeros((pad,), jnp.int32)])

    xh = input_feat.reshape(N_NODES, NC, H).transpose(1, 0, 2)
    tbl = jnp.pad(xh, ((0, 0), (0, NP - N_NODES), (0, 0))).reshape(NC * NP, H)

    sp = src_p.reshape(NS, CHUNKS, LANES)
    gidx = jnp.concatenate([sp, sp + NP], axis=0)
    didx = dst_p.reshape(NS, CHUNKS, LANES)
    zrs = jnp.zeros((ROWS_PER_TILE, H), jnp.float32)

    h0, h1 = _sc_scatter(tbl, gidx, didx, zrs)

    wl0 = W_lin[:, :H]
    wl1 = W_lin[:, H:]
    b = (b_lin + b_loop + bias).reshape(1, D_OUT)
    return _tc_linear(h0, h1, input_feat, wl0, wl1, W_loop, b)


# speedup vs baseline: 4.0928x; 4.0928x over previous
import functools

import jax
import jax.numpy as jnp
from jax import lax
from jax.experimental import pallas as pl
from jax.experimental.pallas import tpu as pltpu
from jax.experimental.pallas import tpu_sc as plsc

N_NODES = 10000
N_EDGES = 160000
D_IN = 256
D_OUT = 256
H = 128
NC = 2
NS = 16
LANES = 128
PER_TILE = -(-N_EDGES // (NS * LANES)) * LANES
CHUNKS = PER_TILE // LANES
E_PAD = PER_TILE * NS
NP = N_NODES + 8
NB = 10240
ROWS_PER_TILE = NB // NS


def _sc_scatter_body(tbl, gidx, didx, zrs, out0, out1, acc, gi_v, di_v, rows, sem):
    c = lax.axis_index("c")
    s = lax.axis_index("s")
    w = c * NS + s
    pltpu.sync_copy(gidx.at[w], gi_v)
    pltpu.sync_copy(didx.at[s], di_v)
    pltpu.sync_copy(zrs, acc.at[pl.ds(s * ROWS_PER_TILE, ROWS_PER_TILE)])
    plsc.subcore_barrier()

    def chunk(j, carry):
        pltpu.async_copy(tbl.at[gi_v.at[j]], rows, sem).wait()
        pltpu.sync_copy(rows, acc.at[di_v.at[j]], add=True)
        return carry

    lax.fori_loop(0, CHUNKS, chunk, 0)
    plsc.subcore_barrier()

    sl = pl.ds(s * ROWS_PER_TILE, ROWS_PER_TILE)

    @pl.when(c == 0)
    def _():
        pltpu.sync_copy(acc.at[sl], out0.at[sl])

    @pl.when(c == 1)
    def _():
        pltpu.sync_copy(acc.at[sl], out1.at[sl])


@functools.partial(
    pl.kernel,
    out_type=(
        jax.ShapeDtypeStruct((NB, H), jnp.float32),
        jax.ShapeDtypeStruct((NB, H), jnp.float32),
    ),
    mesh=plsc.VectorSubcoreMesh(core_axis_name="c", subcore_axis_name="s"),
    scratch_types=[
        pltpu.VMEM_SHARED((NB, H), jnp.float32),
        pltpu.VMEM((CHUNKS, LANES), jnp.int32),
        pltpu.VMEM((CHUNKS, LANES), jnp.int32),
        pltpu.VMEM((LANES, H), jnp.float32),
        pltpu.SemaphoreType.DMA,
    ],
)
def _sc_scatter(tbl, gidx, didx, zrs, out0, out1, acc, gi_v, di_v, rows, sem):
    _sc_scatter_body(tbl, gidx, didx, zrs, out0, out1, acc, gi_v, di_v, rows, sem)


def _mm_body(h0_ref, h1_ref, x_ref, wl0_ref, wl1_ref, wp_ref, b_ref, o_ref):
    dn = (((1,), (1,)), ((), ()))
    acc = lax.dot_general(h0_ref[...], wl0_ref[...], dn,
                          preferred_element_type=jnp.float32)
    acc += lax.dot_general(h1_ref[...], wl1_ref[...], dn,
                           preferred_element_type=jnp.float32)
    acc += lax.dot_general(x_ref[...], wp_ref[...], dn,
                           preferred_element_type=jnp.float32)
    o_ref[...] = acc + b_ref[...]


def _tc_linear(h0, h1, x, wl0, wl1, wp, b):
    blk = 1000
    grid = (N_NODES // blk,)
    return pl.pallas_call(
        _mm_body,
        grid=grid,
        in_specs=[
            pl.BlockSpec((blk, H), lambda i: (i, 0)),
            pl.BlockSpec((blk, H), lambda i: (i, 0)),
            pl.BlockSpec((blk, D_IN), lambda i: (i, 0)),
            pl.BlockSpec((D_OUT, H), lambda i: (0, 0)),
            pl.BlockSpec((D_OUT, H), lambda i: (0, 0)),
            pl.BlockSpec((D_OUT, D_IN), lambda i: (0, 0)),
            pl.BlockSpec((1, D_OUT), lambda i: (0, 0)),
        ],
        out_specs=pl.BlockSpec((blk, D_OUT), lambda i: (i, 0)),
        out_shape=jax.ShapeDtypeStruct((N_NODES, D_OUT), jnp.float32),
    )(h0, h1, x, wl0, wl1, wp, b)


def kernel(input_feat, edge_index, W_lin, b_lin, W_loop, b_loop, bias):
    src = edge_index[0].astype(jnp.int32)
    dst = edge_index[1].astype(jnp.int32)
    pad = E_PAD - N_EDGES
    src_p = jnp.concatenate([src, jnp.full((pad,), N_NODES, jnp.int32)])
    dst_p = jnp.concatenate([dst, jnp.zeros((pad,), jnp.int32)])

    xh = input_feat.reshape(N_NODES, NC, H).transpose(1, 0, 2)
    tbl = jnp.pad(xh, ((0, 0), (0, NP - N_NODES), (0, 0))).reshape(NC * NP, H)

    sp = src_p.reshape(NS, CHUNKS, LANES)
    gidx = jnp.concatenate([sp, sp + NP], axis=0)
    didx = dst_p.reshape(NS, CHUNKS, LANES)
    zrs = jnp.zeros((ROWS_PER_TILE, H), jnp.float32)

    h0, h1 = _sc_scatter(tbl, gidx, didx, zrs)
    h0 = h0[:N_NODES]
    h1 = h1[:N_NODES]

    wl0 = W_lin[:, :H]
    wl1 = W_lin[:, H:]
    b = (b_lin + b_loop + bias).reshape(1, D_OUT)
    return _tc_linear(h0, h1, input_feat, wl0, wl1, W_loop, b)
